# Initial kernel scaffold; baseline (speedup 1.0000x reference)
import jax
import jax.numpy as jnp
from jax.experimental import pallas as pl


def kernel(x, edge_index, W1, b1, W2, b2, Wout, bout):
    raise NotImplementedError("write your pallas kernel here")


# speedup vs baseline: 28.5222x; 28.5222x over previous
import jax
import jax.numpy as jnp
from jax import lax
from jax.experimental import pallas as pl
from jax.experimental.pallas import tpu as pltpu
from jax.experimental.pallas import tpu_sc as plsc

_N = 10000
_E = 320000
_F = 128
_FH = 64
_OUT = 8

_NC = 2
_NS = 16
_CH = 125
_NCHUNK = _E // _CH
_CPT = _NCHUNK // _NS
_NBUF = 4
_N_PAD = 10240
_STRIPE = _N_PAD // _NS
_BLK = 1000


def _mesh():
    return plsc.VectorSubcoreMesh(core_axis_name="c", subcore_axis_name="s")


_SC_PARAMS = pltpu.CompilerParams(use_tc_tiling_on_sc=False)


def _deg_body(dst_hbm, zeros1_hbm, out_hbm, didx, ones_v, acc1, sem_i, sem_s):
    c = lax.axis_index("c")
    s = lax.axis_index("s")
    wid = c * _NS + s
    cpw = _CPT // 2
    ci = pltpu.async_copy(dst_hbm.at[pl.ds(wid * cpw, cpw)], didx, sem_i)
    for k in range(8):
        ones_v[pl.ds(k * 16, 16)] = jnp.ones((16,), jnp.float32)
    pltpu.sync_copy(zeros1_hbm, acc1.at[pl.ds(s * _STRIPE, _STRIPE)])
    ci.wait()
    plsc.subcore_barrier()

    src_view = ones_v.at[pl.ds(0, _CH)]

    def round_(i, carry):
        for b in range(8):
            pltpu.async_copy(src_view, acc1.at[didx.at[8 * i + b]], sem_s,
                             add=True)
        for b in range(8):
            pltpu.make_async_copy(src_view, acc1.at[didx.at[8 * i + b]],
                                  sem_s).wait()
        return carry

    lax.fori_loop(0, cpw // 8, round_, 0)
    plsc.subcore_barrier()
    pltpu.sync_copy(acc1.at[pl.ds(s * _STRIPE, _STRIPE)],
                    out_hbm.at[c].at[pl.ds(s * _STRIPE, _STRIPE)])


@jax.jit
def _deg_call(dst2d, zeros1d):
    return pl.kernel(
        _deg_body,
        out_type=jax.ShapeDtypeStruct((_NC, _N_PAD), jnp.float32),
        mesh=_mesh(),
        compiler_params=_SC_PARAMS,
        scratch_types=[
            pltpu.VMEM((_CPT // 2, _CH), jnp.int32),
            pltpu.VMEM((128,), jnp.float32),
            pltpu.VMEM_SHARED((_N_PAD,), jnp.float32),
            pltpu.SemaphoreType.DMA,
            pltpu.SemaphoreType.DMA,
        ],
    )(dst2d, zeros1d)


def _agg_body(src_hbm, dst_hbm, g_hbm, zeros2_hbm, out_hbm,
              sidx, didx, rows0, rows1, rows2, rows3, acc,
              sg0, sg1, sg2, sg3, ss0, ss1, ss2, ss3, sem_i):
    c = lax.axis_index("c")
    s = lax.axis_index("s")
    row0 = s * _CPT
    ci1 = pltpu.async_copy(src_hbm.at[pl.ds(row0, _CPT)], sidx, sem_i)
    ci2 = pltpu.async_copy(dst_hbm.at[pl.ds(row0, _CPT)], didx, sem_i)
    pltpu.sync_copy(zeros2_hbm, acc.at[pl.ds(s * _STRIPE, _STRIPE)])
    ci1.wait()
    ci2.wait()
    plsc.subcore_barrier()

    g_half = g_hbm.at[c]
    rows = (rows0, rows1, rows2, rows3)
    sg = (sg0, sg1, sg2, sg3)
    ss = (ss0, ss1, ss2, ss3)

    def g_start(b, j):
        pltpu.async_copy(g_half.at[sidx.at[j]], rows[b], sg[b])

    def g_wait(b, j):
        pltpu.make_async_copy(g_half.at[sidx.at[j]], rows[b], sg[b]).wait()

    def s_start(b, j):
        pltpu.async_copy(rows[b], acc.at[didx.at[j]], ss[b], add=True)

    def s_wait(b, j):
        pltpu.make_async_copy(rows[b], acc.at[didx.at[j]], ss[b]).wait()

    for b in range(_NBUF):
        g_start(b, b)

    def round_(i, carry):
        for b in range(_NBUF):
            j = _NBUF * i + b
            g_wait(b, j)
            s_start(b, j)
        for b in range(_NBUF):
            j = _NBUF * i + b
            s_wait(b, j)
            g_start(b, j + _NBUF)
        return carry

    lax.fori_loop(0, _CPT // _NBUF - 1, round_, 0)
    for b in range(_NBUF):
        j = _CPT - _NBUF + b
        g_wait(b, j)
        s_start(b, j)
    for b in range(_NBUF):
        j = _CPT - _NBUF + b
        s_wait(b, j)
    plsc.subcore_barrier()
    pltpu.sync_copy(acc.at[pl.ds(s * _STRIPE, _STRIPE)],
                    out_hbm.at[c].at[pl.ds(s * _STRIPE, _STRIPE)])


@jax.jit
def _agg_call(src2d, dst2d, g, zeros2d):
    return pl.kernel(
        _agg_body,
        out_type=jax.ShapeDtypeStruct((_NC, _N_PAD, _FH), jnp.float32),
        mesh=_mesh(),
        compiler_params=_SC_PARAMS,
        scratch_types=(
            [pltpu.VMEM((_CPT, _CH), jnp.int32),
             pltpu.VMEM((_CPT, _CH), jnp.int32)]
            + [pltpu.VMEM((_CH, _FH), jnp.float32) for _ in range(_NBUF)]
            + [pltpu.VMEM_SHARED((_N_PAD, _FH), jnp.float32)]
            + [pltpu.SemaphoreType.DMA for _ in range(2 * _NBUF + 1)]
        ),
    )(src2d, dst2d, g, zeros2d)


def _split(v):
    return jnp.stack([v[:, :_FH], v[:, _FH:]])


def _prep1_body(x_ref, da_ref, db_ref, w1_ref, g1_ref):
    dinv = lax.rsqrt(da_ref[...] + db_ref[...] + 1.0)
    g = jnp.dot(x_ref[...] * dinv, w1_ref[...],
                preferred_element_type=jnp.float32)
    g1_ref[...] = _split(g)


@jax.jit
def _prep1_call(x, da, db, W1):
    grid = _N // _BLK
    return pl.pallas_call(
        _prep1_body,
        grid=(grid,),
        in_specs=[
            pl.BlockSpec((_BLK, _F), lambda i: (i, 0)),
            pl.BlockSpec((_BLK, 1), lambda i: (i, 0)),
            pl.BlockSpec((_BLK, 1), lambda i: (i, 0)),
            pl.BlockSpec((_F, _F), lambda i: (0, 0)),
        ],
        out_specs=pl.BlockSpec((_NC, _BLK, _FH), lambda i: (0, i, 0)),
        out_shape=jax.ShapeDtypeStruct((_NC, _N, _FH), jnp.float32),
    )(x, da, db, W1)


def _mid_body(agg_ref, g1_ref, da_ref, db_ref, b1_ref, w2_ref, g2_ref):
    dinv = lax.rsqrt(da_ref[...] + db_ref[...] + 1.0)
    aggs = jnp.concatenate([agg_ref[0] + g1_ref[0],
                            agg_ref[1] + g1_ref[1]], axis=1)
    h = jnp.maximum(aggs * dinv + b1_ref[...], 0.0)
    g = jnp.dot(h * dinv, w2_ref[...], preferred_element_type=jnp.float32)
    g2_ref[...] = _split(g)


@jax.jit
def _mid_call(agg, g1, da, db, b1, W2):
    grid = _N // _BLK
    return pl.pallas_call(
        _mid_body,
        grid=(grid,),
        in_specs=[
            pl.BlockSpec((_NC, _BLK, _FH), lambda i: (0, i, 0)),
            pl.BlockSpec((_NC, _BLK, _FH), lambda i: (0, i, 0)),
            pl.BlockSpec((_BLK, 1), lambda i: (i, 0)),
            pl.BlockSpec((_BLK, 1), lambda i: (i, 0)),
            pl.BlockSpec((1, _F), lambda i: (0, 0)),
            pl.BlockSpec((_F, _F), lambda i: (0, 0)),
        ],
        out_specs=pl.BlockSpec((_NC, _BLK, _FH), lambda i: (0, i, 0)),
        out_shape=jax.ShapeDtypeStruct((_NC, _N, _FH), jnp.float32),
    )(agg, g1, da, db, b1, W2)


def _final_body(agg_ref, g2_ref, da_ref, db_ref, b2_ref, wo_ref, bo_ref,
                out_ref):
    dinv = lax.rsqrt(da_ref[...] + db_ref[...] + 1.0)
    aggs = jnp.concatenate([agg_ref[0] + g2_ref[0],
                            agg_ref[1] + g2_ref[1]], axis=1)
    h = jnp.maximum(aggs * dinv + b2_ref[...], 0.0)
    logits = jnp.dot(h, wo_ref[...],
                     preferred_element_type=jnp.float32) + bo_ref[...]
    m = jnp.max(logits, axis=1, keepdims=True)
    e = jnp.exp(logits - m)
    out_ref[...] = e / jnp.sum(e, axis=1, keepdims=True)


@jax.jit
def _final_call(agg, g2, da, db, b2, Wout, bout):
    grid = _N // _BLK
    return pl.pallas_call(
        _final_body,
        grid=(grid,),
        in_specs=[
            pl.BlockSpec((_NC, _BLK, _FH), lambda i: (0, i, 0)),
            pl.BlockSpec((_NC, _BLK, _FH), lambda i: (0, i, 0)),
            pl.BlockSpec((_BLK, 1), lambda i: (i, 0)),
            pl.BlockSpec((_BLK, 1), lambda i: (i, 0)),
            pl.BlockSpec((1, _F), lambda i: (0, 0)),
            pl.BlockSpec((_F, _OUT), lambda i: (0, 0)),
            pl.BlockSpec((1, _OUT), lambda i: (0, 0)),
        ],
        out_specs=pl.BlockSpec((_BLK, _OUT), lambda i: (i, 0)),
        out_shape=jax.ShapeDtypeStruct((_N, _OUT), jnp.float32),
    )(agg, g2, da, db, b2, Wout, bout)


def kernel(x, edge_index, W1, b1, W2, b2, Wout, bout):
    src2d = edge_index[0].reshape(_NCHUNK, _CH)
    dst2d = edge_index[1].reshape(_NCHUNK, _CH)
    zeros1d = jnp.zeros((_STRIPE,), jnp.float32)
    zeros2d = jnp.zeros((_STRIPE, _FH), jnp.float32)

    deg2 = _deg_call(dst2d, zeros1d)
    da = deg2[0, :_N].reshape(_N, 1)
    db = deg2[1, :_N].reshape(_N, 1)

    g1 = _prep1_call(x, da, db, W1)
    agg1 = _agg_call(src2d, dst2d, g1, zeros2d)
    g2 = _mid_call(agg1, g1, da, db, b1.reshape(1, _F), W2)
    agg2 = _agg_call(src2d, dst2d, g2, zeros2d)
    out = _final_call(agg2, g2, da, db, b2.reshape(1, _F), Wout,
                      bout.reshape(1, _OUT))
    return out


# speedup vs baseline: 28.9245x; 1.0141x over previous
import jax
import jax.numpy as jnp
from jax import lax
from jax.experimental import pallas as pl
from jax.experimental.pallas import tpu as pltpu
from jax.experimental.pallas import tpu_sc as plsc

_N = 10000
_E = 320000
_F = 128
_FH = 64
_OUT = 8

_NC = 2
_NS = 16
_CH = 125
_NCHUNK = _E // _CH
_CPT = _NCHUNK // _NS
_NBUF = 5
_N_PAD = 10240
_STRIPE = _N_PAD // _NS
_BLK = 1000


def _mesh():
    return plsc.VectorSubcoreMesh(core_axis_name="c", subcore_axis_name="s")


_SC_PARAMS = pltpu.CompilerParams(use_tc_tiling_on_sc=False)


def _deg_body(dst_hbm, zeros1_hbm, out_hbm, didx, ones_v, acc1, sem_i, sem_s):
    c = lax.axis_index("c")
    s = lax.axis_index("s")
    wid = c * _NS + s
    cpw = _CPT // 2
    ci = pltpu.async_copy(dst_hbm.at[pl.ds(wid * cpw, cpw)], didx, sem_i)
    for k in range(8):
        ones_v[pl.ds(k * 16, 16)] = jnp.ones((16,), jnp.float32)
    pltpu.sync_copy(zeros1_hbm, acc1.at[pl.ds(s * _STRIPE, _STRIPE)])
    ci.wait()
    plsc.subcore_barrier()

    src_view = ones_v.at[pl.ds(0, _CH)]

    def round_(i, carry):
        for b in range(8):
            pltpu.async_copy(src_view, acc1.at[didx.at[8 * i + b]], sem_s,
                             add=True)
        for b in range(8):
            pltpu.make_async_copy(src_view, acc1.at[didx.at[8 * i + b]],
                                  sem_s).wait()
        return carry

    lax.fori_loop(0, cpw // 8, round_, 0)
    plsc.subcore_barrier()
    pltpu.sync_copy(acc1.at[pl.ds(s * _STRIPE, _STRIPE)],
                    out_hbm.at[c].at[pl.ds(s * _STRIPE, _STRIPE)])


@jax.jit
def _deg_call(dst2d, zeros1d):
    return pl.kernel(
        _deg_body,
        out_type=jax.ShapeDtypeStruct((_NC, _N_PAD), jnp.float32),
        mesh=_mesh(),
        compiler_params=_SC_PARAMS,
        scratch_types=[
            pltpu.VMEM((_CPT // 2, _CH), jnp.int32),
            pltpu.VMEM((128,), jnp.float32),
            pltpu.VMEM_SHARED((_N_PAD,), jnp.float32),
            pltpu.SemaphoreType.DMA,
            pltpu.SemaphoreType.DMA,
        ],
    )(dst2d, zeros1d)


def _agg_body(src_hbm, dst_hbm, g_hbm, zeros2_hbm, out_hbm,
              sidx, didx,
              rows0, rows1, rows2, rows3, rows4, acc,
              sg0, sg1, sg2, sg3, sg4,
              ss0, ss1, ss2, ss3, ss4, sem_i):
    c = lax.axis_index("c")
    s = lax.axis_index("s")
    row0 = s * _CPT
    ci1 = pltpu.async_copy(src_hbm.at[pl.ds(row0, _CPT)], sidx, sem_i)
    ci2 = pltpu.async_copy(dst_hbm.at[pl.ds(row0, _CPT)], didx, sem_i)
    pltpu.sync_copy(zeros2_hbm, acc.at[pl.ds(s * _STRIPE, _STRIPE)])
    ci1.wait()
    ci2.wait()
    plsc.subcore_barrier()

    g_half = g_hbm.at[c]
    rows = (rows0, rows1, rows2, rows3, rows4)
    sg = (sg0, sg1, sg2, sg3, sg4)
    ss = (ss0, ss1, ss2, ss3, ss4)

    def g_start(b, j):
        pltpu.async_copy(g_half.at[sidx.at[j]], rows[b], sg[b])

    def g_wait(b, j):
        pltpu.make_async_copy(g_half.at[sidx.at[j]], rows[b], sg[b]).wait()

    def s_start(b, j):
        pltpu.async_copy(rows[b], acc.at[didx.at[j]], ss[b], add=True)

    def s_wait(b, j):
        pltpu.make_async_copy(rows[b], acc.at[didx.at[j]], ss[b]).wait()

    for b in range(_NBUF):
        g_start(b, b)

    def round_(i, carry):
        for b in range(_NBUF):
            j = _NBUF * i + b
            g_wait(b, j)
            s_start(b, j)
        for b in range(_NBUF):
            j = _NBUF * i + b
            s_wait(b, j)
            g_start(b, j + _NBUF)
        return carry

    lax.fori_loop(0, _CPT // _NBUF - 1, round_, 0)
    for b in range(_NBUF):
        j = _CPT - _NBUF + b
        g_wait(b, j)
        s_start(b, j)
    for b in range(_NBUF):
        j = _CPT - _NBUF + b
        s_wait(b, j)
    plsc.subcore_barrier()
    pltpu.sync_copy(acc.at[pl.ds(s * _STRIPE, _STRIPE)],
                    out_hbm.at[c].at[pl.ds(s * _STRIPE, _STRIPE)])


@jax.jit
def _agg_call(src2d, dst2d, g, zeros2d):
    return pl.kernel(
        _agg_body,
        out_type=jax.ShapeDtypeStruct((_NC, _N_PAD, _FH), jnp.float32),
        mesh=_mesh(),
        compiler_params=_SC_PARAMS,
        scratch_types=(
            [pltpu.VMEM((_CPT, _CH), jnp.int32),
             pltpu.VMEM((_CPT, _CH), jnp.int32)]
            + [pltpu.VMEM((_CH, _FH), jnp.float32) for _ in range(_NBUF)]
            + [pltpu.VMEM_SHARED((_N_PAD, _FH), jnp.float32)]
            + [pltpu.SemaphoreType.DMA for _ in range(2 * _NBUF + 1)]
        ),
    )(src2d, dst2d, g, zeros2d)


def _split(v):
    return jnp.stack([v[:, :_FH], v[:, _FH:]])


def _prep1_body(x_ref, da_ref, db_ref, w1_ref, g1_ref):
    dinv = lax.rsqrt(da_ref[...] + db_ref[...] + 1.0)
    g = jnp.dot(x_ref[...] * dinv, w1_ref[...],
                preferred_element_type=jnp.float32)
    g1_ref[...] = _split(g)


@jax.jit
def _prep1_call(x, da, db, W1):
    grid = _N // _BLK
    return pl.pallas_call(
        _prep1_body,
        grid=(grid,),
        in_specs=[
            pl.BlockSpec((_BLK, _F), lambda i: (i, 0)),
            pl.BlockSpec((_BLK, 1), lambda i: (i, 0)),
            pl.BlockSpec((_BLK, 1), lambda i: (i, 0)),
            pl.BlockSpec((_F, _F), lambda i: (0, 0)),
        ],
        out_specs=pl.BlockSpec((_NC, _BLK, _FH), lambda i: (0, i, 0)),
        out_shape=jax.ShapeDtypeStruct((_NC, _N, _FH), jnp.float32),
    )(x, da, db, W1)


def _mid_body(agg_ref, g1_ref, da_ref, db_ref, b1_ref, w2_ref, g2_ref):
    dinv = lax.rsqrt(da_ref[...] + db_ref[...] + 1.0)
    aggs = jnp.concatenate([agg_ref[0] + g1_ref[0],
                            agg_ref[1] + g1_ref[1]], axis=1)
    h = jnp.maximum(aggs * dinv + b1_ref[...], 0.0)
    g = jnp.dot(h * dinv, w2_ref[...], preferred_element_type=jnp.float32)
    g2_ref[...] = _split(g)


@jax.jit
def _mid_call(agg, g1, da, db, b1, W2):
    grid = _N // _BLK
    return pl.pallas_call(
        _mid_body,
        grid=(grid,),
        in_specs=[
            pl.BlockSpec((_NC, _BLK, _FH), lambda i: (0, i, 0)),
            pl.BlockSpec((_NC, _BLK, _FH), lambda i: (0, i, 0)),
            pl.BlockSpec((_BLK, 1), lambda i: (i, 0)),
            pl.BlockSpec((_BLK, 1), lambda i: (i, 0)),
            pl.BlockSpec((1, _F), lambda i: (0, 0)),
            pl.BlockSpec((_F, _F), lambda i: (0, 0)),
        ],
        out_specs=pl.BlockSpec((_NC, _BLK, _FH), lambda i: (0, i, 0)),
        out_shape=jax.ShapeDtypeStruct((_NC, _N, _FH), jnp.float32),
    )(agg, g1, da, db, b1, W2)


def _final_body(agg_ref, g2_ref, da_ref, db_ref, b2_ref, wo_ref, bo_ref,
                out_ref):
    dinv = lax.rsqrt(da_ref[...] + db_ref[...] + 1.0)
    aggs = jnp.concatenate([agg_ref[0] + g2_ref[0],
                            agg_ref[1] + g2_ref[1]], axis=1)
    h = jnp.maximum(aggs * dinv + b2_ref[...], 0.0)
    logits = jnp.dot(h, wo_ref[...],
                     preferred_element_type=jnp.float32) + bo_ref[...]
    m = jnp.max(logits, axis=1, keepdims=True)
    e = jnp.exp(logits - m)
    out_ref[...] = e / jnp.sum(e, axis=1, keepdims=True)


@jax.jit
def _final_call(agg, g2, da, db, b2, Wout, bout):
    grid = _N // _BLK
    return pl.pallas_call(
        _final_body,
        grid=(grid,),
        in_specs=[
            pl.BlockSpec((_NC, _BLK, _FH), lambda i: (0, i, 0)),
            pl.BlockSpec((_NC, _BLK, _FH), lambda i: (0, i, 0)),
            pl.BlockSpec((_BLK, 1), lambda i: (i, 0)),
            pl.BlockSpec((_BLK, 1), lambda i: (i, 0)),
            pl.BlockSpec((1, _F), lambda i: (0, 0)),
            pl.BlockSpec((_F, _OUT), lambda i: (0, 0)),
            pl.BlockSpec((1, _OUT), lambda i: (0, 0)),
        ],
        out_specs=pl.BlockSpec((_BLK, _OUT), lambda i: (i, 0)),
        out_shape=jax.ShapeDtypeStruct((_N, _OUT), jnp.float32),
    )(agg, g2, da, db, b2, Wout, bout)


def kernel(x, edge_index, W1, b1, W2, b2, Wout, bout):
    src2d = edge_index[0].reshape(_NCHUNK, _CH)
    dst2d = edge_index[1].reshape(_NCHUNK, _CH)
    zeros1d = jnp.zeros((_STRIPE,), jnp.float32)
    zeros2d = jnp.zeros((_STRIPE, _FH), jnp.float32)

    deg2 = _deg_call(dst2d, zeros1d)
    da = deg2[0, :_N].reshape(_N, 1)
    db = deg2[1, :_N].reshape(_N, 1)

    g1 = _prep1_call(x, da, db, W1)
    agg1 = _agg_call(src2d, dst2d, g1, zeros2d)
    g2 = _mid_call(agg1, g1, da, db, b1.reshape(1, _F), W2)
    agg2 = _agg_call(src2d, dst2d, g2, zeros2d)
    out = _final_call(agg2, g2, da, db, b2.reshape(1, _F), Wout,
                      bout.reshape(1, _OUT))
    return out


# speedup vs baseline: 29.7156x; 1.0273x over previous
import jax
import jax.numpy as jnp
from jax import lax
from jax.experimental import pallas as pl
from jax.experimental.pallas import tpu as pltpu
from jax.experimental.pallas import tpu_sc as plsc

_N = 10000
_E = 320000
_F = 128
_FH = 64
_OUT = 8

_NC = 2
_NS = 16
_CH = 125
_NCHUNK = _E // _CH
_CPT = _NCHUNK // _NS
_NBUF = 5
_N_PAD = 10240
_STRIPE = _N_PAD // _NS
_BLK = 2000


def _mesh():
    return plsc.VectorSubcoreMesh(core_axis_name="c", subcore_axis_name="s")


_SC_PARAMS = pltpu.CompilerParams(use_tc_tiling_on_sc=False)


def _deg_body(edge_hbm, zeros1_hbm, ones_hbm, out_hbm,
              didx, ones_v, acc1, sem_i, sem_s):
    c = lax.axis_index("c")
    s = lax.axis_index("s")
    wid = c * _NS + s
    cpw = _CPT // 2
    ci = pltpu.async_copy(edge_hbm.at[1].at[pl.ds(wid * cpw, cpw)], didx,
                          sem_i)
    pltpu.sync_copy(ones_hbm, ones_v)
    pltpu.sync_copy(zeros1_hbm, acc1.at[pl.ds(s * _STRIPE, _STRIPE)])
    ci.wait()
    plsc.subcore_barrier()

    src_view = ones_v.at[pl.ds(0, _CH)]

    def round_(i, carry):
        for b in range(8):
            pltpu.async_copy(src_view, acc1.at[didx.at[8 * i + b]], sem_s,
                             add=True)
        for b in range(8):
            pltpu.make_async_copy(src_view, acc1.at[didx.at[8 * i + b]],
                                  sem_s).wait()
        return carry

    lax.fori_loop(0, cpw // 8, round_, 0)
    plsc.subcore_barrier()
    pltpu.sync_copy(acc1.at[pl.ds(s * _STRIPE, _STRIPE)],
                    out_hbm.at[c].at[pl.ds(s * _STRIPE, _STRIPE)])


@jax.jit
def _deg_call(edge2d, zeros1, ones1):
    return pl.kernel(
        _deg_body,
        out_type=jax.ShapeDtypeStruct((_NC, _N_PAD, 1), jnp.float32),
        mesh=_mesh(),
        compiler_params=_SC_PARAMS,
        scratch_types=[
            pltpu.VMEM((_CPT // 2, _CH), jnp.int32),
            pltpu.VMEM((128, 1), jnp.float32),
            pltpu.VMEM_SHARED((_N_PAD, 1), jnp.float32),
            pltpu.SemaphoreType.DMA,
            pltpu.SemaphoreType.DMA,
        ],
    )(edge2d, zeros1, ones1)


def _agg_body(edge_hbm, g_hbm, zeros2_hbm, out_hbm,
              sidx, didx,
              rows0, rows1, rows2, rows3, rows4, acc,
              sg0, sg1, sg2, sg3, sg4,
              ss0, ss1, ss2, ss3, ss4, sem_i):
    c = lax.axis_index("c")
    s = lax.axis_index("s")
    row0 = s * _CPT
    ci1 = pltpu.async_copy(edge_hbm.at[0].at[pl.ds(row0, _CPT)], sidx, sem_i)
    ci2 = pltpu.async_copy(edge_hbm.at[1].at[pl.ds(row0, _CPT)], didx, sem_i)
    pltpu.sync_copy(zeros2_hbm, acc.at[pl.ds(s * _STRIPE, _STRIPE)])
    ci1.wait()
    ci2.wait()
    plsc.subcore_barrier()

    g_half = g_hbm.at[c]
    rows = (rows0, rows1, rows2, rows3, rows4)
    sg = (sg0, sg1, sg2, sg3, sg4)
    ss = (ss0, ss1, ss2, ss3, ss4)

    def g_start(b, j):
        pltpu.async_copy(g_half.at[sidx.at[j]], rows[b], sg[b])

    def g_wait(b, j):
        pltpu.make_async_copy(g_half.at[sidx.at[j]], rows[b], sg[b]).wait()

    def s_start(b, j):
        pltpu.async_copy(rows[b], acc.at[didx.at[j]], ss[b], add=True)

    def s_wait(b, j):
        pltpu.make_async_copy(rows[b], acc.at[didx.at[j]], ss[b]).wait()

    for b in range(_NBUF):
        g_start(b, b)

    def round_(i, carry):
        for b in range(_NBUF):
            j = _NBUF * i + b
            g_wait(b, j)
            s_start(b, j)
        for b in range(_NBUF):
            j = _NBUF * i + b
            s_wait(b, j)
            g_start(b, j + _NBUF)
        return carry

    lax.fori_loop(0, _CPT // _NBUF - 1, round_, 0)
    for b in range(_NBUF):
        j = _CPT - _NBUF + b
        g_wait(b, j)
        s_start(b, j)
    for b in range(_NBUF):
        j = _CPT - _NBUF + b
        s_wait(b, j)
    plsc.subcore_barrier()
    pltpu.sync_copy(acc.at[pl.ds(s * _STRIPE, _STRIPE)],
                    out_hbm.at[c].at[pl.ds(s * _STRIPE, _STRIPE)])


@jax.jit
def _agg_call(edge2d, g, zeros2d):
    return pl.kernel(
        _agg_body,
        out_type=jax.ShapeDtypeStruct((_NC, _N_PAD, _FH), jnp.float32),
        mesh=_mesh(),
        compiler_params=_SC_PARAMS,
        scratch_types=(
            [pltpu.VMEM((_CPT, _CH), jnp.int32),
             pltpu.VMEM((_CPT, _CH), jnp.int32)]
            + [pltpu.VMEM((_CH, _FH), jnp.float32) for _ in range(_NBUF)]
            + [pltpu.VMEM_SHARED((_N_PAD, _FH), jnp.float32)]
            + [pltpu.SemaphoreType.DMA for _ in range(2 * _NBUF + 1)]
        ),
    )(edge2d, g, zeros2d)


def _split(v):
    return jnp.stack([v[:, :_FH], v[:, _FH:]])


def _dinv(da_ref, db_ref):
    return lax.rsqrt(da_ref[0] + db_ref[0] + 1.0)


def _prep1_body(x_ref, da_ref, db_ref, w1_ref, g1_ref):
    g = jnp.dot(x_ref[...] * _dinv(da_ref, db_ref), w1_ref[...],
                preferred_element_type=jnp.float32)
    g1_ref[...] = _split(g)


@jax.jit
def _prep1_call(x, deg2, W1):
    grid = _N // _BLK
    return pl.pallas_call(
        _prep1_body,
        grid=(grid,),
        in_specs=[
            pl.BlockSpec((_BLK, _F), lambda i: (i, 0)),
            pl.BlockSpec((1, _BLK, 1), lambda i: (0, i, 0)),
            pl.BlockSpec((1, _BLK, 1), lambda i: (1, i, 0)),
            pl.BlockSpec((_F, _F), lambda i: (0, 0)),
        ],
        out_specs=pl.BlockSpec((_NC, _BLK, _FH), lambda i: (0, i, 0)),
        out_shape=jax.ShapeDtypeStruct((_NC, _N, _FH), jnp.float32),
    )(x, deg2, deg2, W1)


def _mid_body(agg_ref, g1_ref, da_ref, db_ref, b1_ref, w2_ref, g2_ref):
    dinv = _dinv(da_ref, db_ref)
    aggs = jnp.concatenate([agg_ref[0] + g1_ref[0],
                            agg_ref[1] + g1_ref[1]], axis=1)
    h = jnp.maximum(aggs * dinv + b1_ref[...], 0.0)
    g = jnp.dot(h * dinv, w2_ref[...], preferred_element_type=jnp.float32)
    g2_ref[...] = _split(g)


@jax.jit
def _mid_call(agg, g1, deg2, b1, W2):
    grid = _N // _BLK
    return pl.pallas_call(
        _mid_body,
        grid=(grid,),
        in_specs=[
            pl.BlockSpec((_NC, _BLK, _FH), lambda i: (0, i, 0)),
            pl.BlockSpec((_NC, _BLK, _FH), lambda i: (0, i, 0)),
            pl.BlockSpec((1, _BLK, 1), lambda i: (0, i, 0)),
            pl.BlockSpec((1, _BLK, 1), lambda i: (1, i, 0)),
            pl.BlockSpec((_F,), lambda i: (0,)),
            pl.BlockSpec((_F, _F), lambda i: (0, 0)),
        ],
        out_specs=pl.BlockSpec((_NC, _BLK, _FH), lambda i: (0, i, 0)),
        out_shape=jax.ShapeDtypeStruct((_NC, _N, _FH), jnp.float32),
    )(agg, g1, deg2, deg2, b1, W2)


def _final_body(agg_ref, g2_ref, da_ref, db_ref, b2_ref, wo_ref, bo_ref,
                out_ref):
    dinv = _dinv(da_ref, db_ref)
    aggs = jnp.concatenate([agg_ref[0] + g2_ref[0],
                            agg_ref[1] + g2_ref[1]], axis=1)
    h = jnp.maximum(aggs * dinv + b2_ref[...], 0.0)
    logits = jnp.dot(h, wo_ref[...],
                     preferred_element_type=jnp.float32) + bo_ref[...]
    m = jnp.max(logits, axis=1, keepdims=True)
    e = jnp.exp(logits - m)
    out_ref[...] = e / jnp.sum(e, axis=1, keepdims=True)


@jax.jit
def _final_call(agg, g2, deg2, b2, Wout, bout):
    grid = _N // _BLK
    return pl.pallas_call(
        _final_body,
        grid=(grid,),
        in_specs=[
            pl.BlockSpec((_NC, _BLK, _FH), lambda i: (0, i, 0)),
            pl.BlockSpec((_NC, _BLK, _FH), lambda i: (0, i, 0)),
            pl.BlockSpec((1, _BLK, 1), lambda i: (0, i, 0)),
            pl.BlockSpec((1, _BLK, 1), lambda i: (1, i, 0)),
            pl.BlockSpec((_F,), lambda i: (0,)),
            pl.BlockSpec((_F, _OUT), lambda i: (0, 0)),
            pl.BlockSpec((_OUT,), lambda i: (0,)),
        ],
        out_specs=pl.BlockSpec((_BLK, _OUT), lambda i: (i, 0)),
        out_shape=jax.ShapeDtypeStruct((_N, _OUT), jnp.float32),
    )(agg, g2, deg2, deg2, b2, Wout, bout)


def kernel(x, edge_index, W1, b1, W2, b2, Wout, bout):
    edge2d = edge_index.reshape(2, _NCHUNK, _CH)
    zeros1 = jnp.zeros((_STRIPE, 1), jnp.float32)
    ones1 = jnp.ones((128, 1), jnp.float32)
    zeros2d = jnp.zeros((_STRIPE, _FH), jnp.float32)

    deg2 = _deg_call(edge2d, zeros1, ones1)
    g1 = _prep1_call(x, deg2, W1)
    agg1 = _agg_call(edge2d, g1, zeros2d)
    g2 = _mid_call(agg1, g1, deg2, b1, W2)
    agg2 = _agg_call(edge2d, g2, zeros2d)
    out = _final_call(agg2, g2, deg2, b2, Wout, bout)
    return out
